# Initial kernel scaffold; baseline (speedup 1.0000x reference)
import jax
import jax.numpy as jnp
from jax.experimental import pallas as pl


def kernel(pts_neck_output, W_sh, b_sh, W_hm, b_hm, W_reg, b_reg, W_hei, b_hei, W_dim, b_dim, W_rot, b_rot):
    raise NotImplementedError("write your pallas kernel here")


# speedup vs baseline: 1.1308x; 1.1308x over previous
import functools

import jax
import jax.numpy as jnp
from jax.experimental import pallas as pl
from jax.experimental.pallas import tpu as pltpu

_C = 256
_F = 64
_NC = 10
_H = 256
_W = 256
_NPIX = _H * _W
_CHUNK = 8192
_NSTEPS = _NPIX // _CHUNK


def _dense_body(x_ref, wsh_ref, bsh_ref, wcat_ref, bcat_ref, hm_ref, heads_ref):
    x = x_ref[...]
    feat = jax.lax.dot_general(
        x, wsh_ref[...], (((0,), (0,)), ((), ())),
        preferred_element_type=jnp.float32)
    feat = jax.nn.relu(feat + bsh_ref[...][None, :])
    out = jax.lax.dot_general(
        feat, wcat_ref[...], (((1,), (0,)), ((), ())),
        preferred_element_type=jnp.float32)
    out = out + bcat_ref[...][None, :]
    hm_ref[...] = jax.nn.sigmoid(out[:, :_NC])
    heads_ref[...] = out[:, _NC:]


def _dense_stage(x2d, W_sh, b_sh, W_cat, b_cat):
    return pl.pallas_call(
        _dense_body,
        grid=(_NSTEPS,),
        in_specs=[
            pl.BlockSpec((_C, _CHUNK), lambda i: (0, i)),
            pl.BlockSpec((_C, _F), lambda i: (0, 0)),
            pl.BlockSpec((_F,), lambda i: (0,)),
            pl.BlockSpec((_F, 18), lambda i: (0, 0)),
            pl.BlockSpec((18,), lambda i: (0,)),
        ],
        out_specs=[
            pl.BlockSpec((_CHUNK, _NC), lambda i: (i, 0)),
            pl.BlockSpec((_CHUNK, 8), lambda i: (i, 0)),
        ],
        out_shape=[
            jax.ShapeDtypeStruct((_NPIX, _NC), jnp.float32),
            jax.ShapeDtypeStruct((_NPIX, 8), jnp.float32),
        ],
    )(x2d, W_sh, b_sh, W_cat, b_cat)


def kernel(pts_neck_output, W_sh, b_sh, W_hm, b_hm, W_reg, b_reg,
           W_hei, b_hei, W_dim, b_dim, W_rot, b_rot):
    x2d = pts_neck_output.reshape(_C, _NPIX)
    W_cat = jnp.concatenate([W_hm, W_reg, W_hei, W_dim, W_rot], axis=1)
    b_cat = jnp.concatenate([b_hm, b_reg, b_hei, b_dim, b_rot], axis=0)
    hm_pm, heads_pm = _dense_stage(x2d, W_sh, b_sh, W_cat, b_cat)

    hm = hm_pm.T.reshape(1, _NC, _H, _W)
    B = 1
    hmax = jax.lax.reduce_window(hm, -jnp.inf, jax.lax.max,
                                 (1, 1, 3, 3), (1, 1, 1, 1), 'SAME')
    keep = (hmax == hm).astype(hm.dtype)
    scores_all = (hm * keep).reshape(B, _NC * _NPIX)
    K = 500
    scores, idx = jax.lax.top_k(scores_all, K)
    labels = idx // _NPIX
    spatial = idx % _NPIX
    ys = (spatial // _W).astype(jnp.float32)
    xs = (spatial % _W).astype(jnp.float32)

    g = heads_pm[spatial[0]]
    regk = g[None, :, 0:2]
    heik = g[None, :, 2:3]
    dimk = jnp.exp(jnp.clip(g[None, :, 3:6], -5.0, 5.0))
    rotk = g[None, :, 6:8]

    voxel, factor, pcr = 0.2, 4.0, -51.2
    xs_w = (xs + regk[..., 0]) * factor * voxel + pcr
    ys_w = (ys + regk[..., 1]) * factor * voxel + pcr
    rot_a = jnp.arctan2(rotk[..., 0], rotk[..., 1])

    bboxes = jnp.stack([xs_w, ys_w, heik[..., 0],
                        dimk[..., 0], dimk[..., 1], dimk[..., 2], rot_a],
                       axis=-1)
    return bboxes, scores, labels
